# baseline (speedup 1.0000x reference)
import functools

import jax
import jax.numpy as jnp
from jax import lax
from jax.experimental import pallas as pl
from jax.experimental.pallas import tpu as pltpu
from jax.experimental.pallas import tpu_sc as plsc

B = 16384
RANK = 16
ROWS_PER_LINE = 8
LINE = ROWS_PER_LINE * RANK

_info = plsc.get_sparse_core_info()
NC = _info.num_cores
NS = _info.num_subcores
L = _info.num_lanes
NW = NC * NS
BPW = B // NW
CHUNK = 128
NCHUNK = BPW // CHUNK
BLKS = CHUNK // L

_mesh = plsc.VectorSubcoreMesh(core_axis_name="c", subcore_axis_name="s")


@functools.partial(
    pl.kernel,
    mesh=_mesh,
    out_type=(
        jax.ShapeDtypeStruct((B,), jnp.float32),
        jax.ShapeDtypeStruct((B,), jnp.float32),
    ),
    scratch_types=[
        pltpu.VMEM((BPW,), jnp.int32),
        pltpu.VMEM((BPW,), jnp.int32),
        pltpu.VMEM((BPW,), jnp.int32),
        pltpu.VMEM((BPW,), jnp.int32),
        pltpu.VMEM((BPW,), jnp.int32),
        pltpu.VMEM((BPW,), jnp.int32),
        pltpu.VMEM((CHUNK, LINE), jnp.float32),
        pltpu.VMEM((CHUNK, LINE), jnp.float32),
        pltpu.VMEM((CHUNK, LINE), jnp.float32),
        pltpu.VMEM((CHUNK, LINE), jnp.float32),
        pltpu.VMEM((CHUNK, LINE), jnp.float32),
        pltpu.VMEM((CHUNK, LINE), jnp.float32),
        pltpu.VMEM((BPW,), jnp.float32),
        pltpu.VMEM((BPW,), jnp.float32),
        pltpu.SemaphoreType.DMA,
        pltpu.SemaphoreType.DMA,
    ],
    compiler_params=pltpu.CompilerParams(needs_layout_passes=False),
)
def _bpr_sc(ulin_hbm, plin_hbm, nlin_hbm, ucol_hbm, pcol_hbm, ncol_hbm,
            utab_hbm, itab_hbm, outp_hbm, outn_hbm,
            ulin_v, plin_v, nlin_v, ucol_v, pcol_v, ncol_v,
            ua_v, ub_v, pa_v, pb_v, na_v, nb_v,
            outp_v, outn_v, semA, semB):
    wid = lax.axis_index("s") * NC + lax.axis_index("c")
    base0 = wid * BPW

    sl = pl.ds(base0, BPW)
    pltpu.sync_copy(ulin_hbm.at[sl], ulin_v)
    pltpu.sync_copy(plin_hbm.at[sl], plin_v)
    pltpu.sync_copy(nlin_hbm.at[sl], nlin_v)
    pltpu.sync_copy(ucol_hbm.at[sl], ucol_v)
    pltpu.sync_copy(pcol_hbm.at[sl], pcol_v)
    pltpu.sync_copy(ncol_hbm.at[sl], ncol_v)

    bufs = [(ua_v, pa_v, na_v, semA), (ub_v, pb_v, nb_v, semB)]

    def fire(j):
        ub, pb, nb, sem = bufs[j % 2]
        s = pl.ds(j * CHUNK, CHUNK)
        return [
            pltpu.async_copy(utab_hbm.at[ulin_v.at[s]], ub, sem),
            pltpu.async_copy(itab_hbm.at[plin_v.at[s]], pb, sem),
            pltpu.async_copy(itab_hbm.at[nlin_v.at[s]], nb, sem),
        ]

    iota = lax.iota(jnp.int32, L)
    inflight = fire(0)
    for j in range(NCHUNK):
        if j + 1 < NCHUNK:
            nxt = fire(j + 1)
        else:
            nxt = None
        for c in inflight:
            c.wait()
        inflight = nxt

        ub, pb, nb, _ = bufs[j % 2]

        def blk_body(b, carry):
            gb = j * CHUNK + b * L
            lrows = b * L + iota
            uc = ucol_v[pl.ds(gb, L)]
            pc = pcol_v[pl.ds(gb, L)]
            nc = ncol_v[pl.ds(gb, L)]
            accp = jnp.zeros((L,), jnp.float32)
            accn = jnp.zeros((L,), jnp.float32)
            for k in range(RANK):
                u = plsc.load_gather(ub, [lrows, uc + k])
                p = plsc.load_gather(pb, [lrows, pc + k])
                n = plsc.load_gather(nb, [lrows, nc + k])
                accp = accp + u * p
                accn = accn + u * n
            outp_v[pl.ds(gb, L)] = accp
            outn_v[pl.ds(gb, L)] = accn
            return carry

        lax.fori_loop(0, BLKS, blk_body, 0)

    pltpu.sync_copy(outp_v, outp_hbm.at[sl])
    pltpu.sync_copy(outn_v, outn_hbm.at[sl])


def kernel(user_ids, pos_items, neg_items, user_emb, item_emb):
    uids = user_ids.astype(jnp.int32)
    pids = pos_items.astype(jnp.int32)
    nids = neg_items.astype(jnp.int32)
    utab = user_emb.reshape(-1, LINE)
    itab = item_emb.reshape(-1, LINE)
    return _bpr_sc(
        uids >> 3, pids >> 3, nids >> 3,
        (uids & 7) << 4, (pids & 7) << 4, (nids & 7) << 4,
        utab, itab)
